# baseline (speedup 1.0000x reference)
import functools

import jax
import jax.numpy as jnp
from jax import lax
from jax.experimental import pallas as pl
from jax.experimental.pallas import tpu as pltpu
from jax.experimental.pallas import tpu_sc as plsc

_B = 32
_F = 3
_P = 1024
_R = _B * _F
_N2 = int(0.9 * _P)
_NV = _P // 16


def _threefry2x32(ks0, ks1, x0, x1):
    ks2 = ks0 ^ ks1 ^ jnp.int32(0x1BD11BDA)
    ks = (ks0, ks1, ks2)
    rots = ((13, 15, 26, 6), (17, 29, 16, 24))
    x0 = x0 + ks0
    x1 = x1 + ks1
    for g in range(5):
        for r in rots[g % 2]:
            x0 = x0 + x1
            x1 = (x1 << jnp.int32(r)) | lax.shift_right_logical(x1, jnp.int32(32 - r))
            x1 = x1 ^ x0
        x0 = x0 + ks[(g + 1) % 3]
        x1 = x1 + ks[(g + 2) % 3] + jnp.int32(g + 1)
    return x0, x1


def _sc_body(out_hbm, m_v, o_v):
    cid = lax.axis_index("c")
    sid = lax.axis_index("s")
    w = sid * 2 + cid

    zero = jnp.int32(0)
    one = jnp.int32(1)
    vzero = jnp.zeros((16,), jnp.int32)
    k1h, k1l = _threefry2x32(zero, jnp.int32(42), zero, zero)
    k2h, k2l = _threefry2x32(zero, jnp.int32(42), zero, one)
    u0, u1 = _threefry2x32(k1h, k1l, zero, w)
    n1 = lax.shift_right_logical(u0 ^ u1, jnp.int32(22))

    def row_body(k, _):
        r = 3 * w + k
        n = jnp.where(k == 0, n1, jnp.where(k == 1, jnp.int32(_N2), jnp.int32(_P) - n1))

        def gen(v, _c):
            lane = lax.iota(jnp.int32, 16)
            cnt = r * jnp.int32(_P) + v * jnp.int32(16) + lane
            y0, y1 = _threefry2x32(k2h, k2l, zero, cnt)
            m_v[pl.ds(v * 16, 16)] = lax.shift_right_logical(y0 ^ y1, jnp.int32(9))
            return _c
        lax.fori_loop(0, _NV, gen, zero, unroll=8)

        pref = zero
        rem = n
        for bit in range(22, -1, -1):
            tgt = pref << one

            def cpass(v, acc, _bit=bit, _tgt=tgt):
                mv = m_v[pl.ds(v * 16, 16)]
                hit = lax.shift_right_logical(mv, jnp.int32(_bit)) == _tgt
                return acc + hit.astype(jnp.int32)
            accv = lax.fori_loop(0, _NV, cpass, vzero, unroll=8)
            c0 = jnp.sum(accv)
            go1 = rem > c0
            pref = tgt | go1.astype(jnp.int32)
            rem = rem - jnp.where(go1, c0, zero)
        t = pref

        def tpass(v, accs):
            acc_e, acc_j = accs
            lane = lax.iota(jnp.int32, 16)
            mv = m_v[pl.ds(v * 16, 16)]
            eq = (mv == t).astype(jnp.int32)
            jj1 = v * jnp.int32(16) + lane + one
            return (acc_e + eq, acc_j + eq * jj1)
        acc_e, acc_j = lax.fori_loop(0, _NV, tpass, (vzero, vzero), unroll=8)
        c_eq = jnp.sum(acc_e)

        def unique_j(_):
            return jnp.sum(acc_j) - one

        def radix_j(_):
            prefj = zero
            remj = rem
            for bit in range(9, -1, -1):
                tgtj = prefj << one

                def jpass(v, acc, _bit=bit, _tgtj=tgtj):
                    lane = lax.iota(jnp.int32, 16)
                    mv = m_v[pl.ds(v * 16, 16)]
                    jj = v * jnp.int32(16) + lane
                    hit = (mv == t) & (lax.shift_right_logical(jj, jnp.int32(_bit)) == _tgtj)
                    return acc + hit.astype(jnp.int32)
                accj = lax.fori_loop(0, _NV, jpass, vzero, unroll=8)
                cj = jnp.sum(accj)
                go = remj > cj
                prefj = tgtj | go.astype(jnp.int32)
                remj = remj - jnp.where(go, cj, zero)
            return prefj

        j_t = lax.cond(c_eq == one, unique_j, radix_j, zero)
        t_eff = jnp.where(n > 0, t, jnp.int32(-1))
        j_eff = jnp.where(n > 0, j_t, jnp.int32(-1))

        def emit(v, _c):
            lane = lax.iota(jnp.int32, 16)
            mv = m_v[pl.ds(v * 16, 16)]
            jj = v * jnp.int32(16) + lane
            mask = (mv < t_eff) | ((mv == t_eff) & (jj <= j_eff))
            o_v[pl.ds(v * 16, 16)] = mask.astype(jnp.int32)
            return _c
        lax.fori_loop(0, _NV, emit, zero, unroll=8)
        pltpu.sync_copy(o_v, out_hbm.at[pl.ds(r * _P, _P)])
        return zero

    lax.fori_loop(0, 3, row_body, zero)


def kernel(x):
    sc_fn = functools.partial(
        pl.kernel,
        out_type=jax.ShapeDtypeStruct((_R * _P,), jnp.int32),
        mesh=plsc.VectorSubcoreMesh(core_axis_name="c", subcore_axis_name="s"),
        compiler_params=pltpu.CompilerParams(needs_layout_passes=False),
        scratch_types=[
            pltpu.VMEM((_P,), jnp.int32),
            pltpu.VMEM((_P,), jnp.int32),
        ],
    )(_sc_body)
    flat = sc_fn()
    return flat.reshape(_B, _F * _P).astype(jnp.bool_)


# speedup vs baseline: 1.2524x; 1.2524x over previous
import functools

import jax
import jax.numpy as jnp
from jax import lax
from jax.experimental import pallas as pl
from jax.experimental.pallas import tpu as pltpu
from jax.experimental.pallas import tpu_sc as plsc

_B = 32
_F = 3
_P = 1024
_R = _B * _F
_N2 = int(0.9 * _P)
_NV = _P // 16
_SC_ROWS = 32
_TC_ROWS = _R - _SC_ROWS


def _threefry2x32(ks0, ks1, x0, x1, shr):
    ks2 = ks0 ^ ks1 ^ _c(ks0, 0x1BD11BDA)
    ks = (ks0, ks1, ks2)
    rots = ((13, 15, 26, 6), (17, 29, 16, 24))
    x0 = x0 + ks0
    x1 = x1 + ks1
    for g in range(5):
        for r in rots[g % 2]:
            x0 = x0 + x1
            x1 = (x1 << _c(x1, r)) | shr(x1, 32 - r)
            x1 = x1 ^ x0
        x0 = x0 + ks[(g + 1) % 3]
        x1 = x1 + ks[(g + 2) % 3] + _c(x1, g + 1)
    return x0, x1


def _c(like, v):
    return jnp.asarray(v, dtype=like.dtype if hasattr(like, "dtype") else jnp.int32)


def _shr_u(x, r):
    return x >> jnp.uint32(r)


def _shr_s(x, r):
    return lax.shift_right_logical(x, jnp.int32(r))


def _sc_body(out_hbm, m_v, o_v):
    cid = lax.axis_index("c")
    sid = lax.axis_index("s")
    w = sid * 2 + cid

    zero = jnp.int32(0)
    one = jnp.int32(1)
    vzero = jnp.zeros((16,), jnp.int32)
    k1h, k1l = _threefry2x32(zero, jnp.int32(42), zero, zero, _shr_s)
    k2h, k2l = _threefry2x32(zero, jnp.int32(42), zero, one, _shr_s)

    r = w
    b = lax.shift_right_logical(r * jnp.int32(21846), jnp.int32(16))
    f = r - 3 * b
    u0, u1 = _threefry2x32(k1h, k1l, zero, b, _shr_s)
    n1 = lax.shift_right_logical(u0 ^ u1, jnp.int32(22))
    n = jnp.where(f == 0, n1, jnp.where(f == 1, jnp.int32(_N2), jnp.int32(_P) - n1))

    def gen(v, _cc):
        lane = lax.iota(jnp.int32, 16)
        cnt = r * jnp.int32(_P) + v * jnp.int32(16) + lane
        y0, y1 = _threefry2x32(k2h, k2l, zero, cnt, _shr_s)
        m_v[pl.ds(v * 16, 16)] = lax.shift_right_logical(y0 ^ y1, jnp.int32(9))
        return _cc
    lax.fori_loop(0, _NV, gen, zero, unroll=8)

    pref = zero
    rem = n
    for bit in range(22, -1, -1):
        tgt = pref << one

        def cpass(v, acc, _bit=bit, _tgt=tgt):
            mv = m_v[pl.ds(v * 16, 16)]
            hit = lax.shift_right_logical(mv, jnp.int32(_bit)) == _tgt
            return acc + hit.astype(jnp.int32)
        accv = lax.fori_loop(0, _NV, cpass, vzero, unroll=8)
        c0 = jnp.sum(accv)
        go1 = rem > c0
        pref = tgt | go1.astype(jnp.int32)
        rem = rem - jnp.where(go1, c0, zero)
    t = pref

    def tpass(v, accs):
        acc_e, acc_j = accs
        lane = lax.iota(jnp.int32, 16)
        mv = m_v[pl.ds(v * 16, 16)]
        eq = (mv == t).astype(jnp.int32)
        jj1 = v * jnp.int32(16) + lane + one
        return (acc_e + eq, acc_j + eq * jj1)
    acc_e, acc_j = lax.fori_loop(0, _NV, tpass, (vzero, vzero), unroll=8)
    c_eq = jnp.sum(acc_e)

    def unique_j(_):
        return jnp.sum(acc_j) - one

    def radix_j(_):
        prefj = zero
        remj = rem
        for bit in range(9, -1, -1):
            tgtj = prefj << one

            def jpass(v, acc, _bit=bit, _tgtj=tgtj):
                lane = lax.iota(jnp.int32, 16)
                mv = m_v[pl.ds(v * 16, 16)]
                jj = v * jnp.int32(16) + lane
                hit = (mv == t) & (lax.shift_right_logical(jj, jnp.int32(_bit)) == _tgtj)
                return acc + hit.astype(jnp.int32)
            accj = lax.fori_loop(0, _NV, jpass, vzero, unroll=8)
            cj = jnp.sum(accj)
            go = remj > cj
            prefj = tgtj | go.astype(jnp.int32)
            remj = remj - jnp.where(go, cj, zero)
        return prefj

    j_t = lax.cond(c_eq == one, unique_j, radix_j, zero)
    t_eff = jnp.where(n > 0, t, jnp.int32(-1))
    j_eff = jnp.where(n > 0, j_t, jnp.int32(-1))

    def emit(v, _cc):
        lane = lax.iota(jnp.int32, 16)
        mv = m_v[pl.ds(v * 16, 16)]
        jj = v * jnp.int32(16) + lane
        mask = (mv < t_eff) | ((mv == t_eff) & (jj <= j_eff))
        o_v[pl.ds(v * 16, 16)] = mask.astype(jnp.int32)
        return _cc
    lax.fori_loop(0, _NV, emit, zero, unroll=8)
    pltpu.sync_copy(o_v, out_hbm.at[pl.ds(r * _P, _P)])


def _tc_select_rank(keys, nbits, rem, cand):
    pref = jnp.zeros_like(rem)
    for bit in range(nbits - 1, -1, -1):
        ms = keys >> bit
        match = (ms >> 1) == pref
        if cand is not None:
            match = match & cand
        in0 = match & ((ms & 1) == 0)
        c0 = jnp.sum(in0.astype(jnp.int32), axis=1, keepdims=True)
        go1 = rem > c0
        pref = (pref << 1) | go1.astype(jnp.int32)
        rem = rem - jnp.where(go1, c0, 0)
    return pref, rem


def _tc_body(out_ref):
    col8 = lax.broadcasted_iota(jnp.uint32, (8, 128), 1)
    row8 = lax.broadcasted_iota(jnp.uint32, (8, 128), 0)
    s0, s1 = _threefry2x32(jnp.uint32(0), jnp.uint32(42),
                           jnp.zeros((8, 128), jnp.uint32), col8, _shr_u)
    top = row8 == jnp.uint32(0)
    sel_a = top & (col8 == jnp.uint32(0))
    sel_b = top & (col8 == jnp.uint32(1))

    def _pick(sel, v):
        vi = lax.bitcast_convert_type(v, jnp.int32)
        s = jnp.sum(jnp.where(sel, vi, 0))
        return lax.bitcast_convert_type(s, jnp.uint32)

    k1h = _pick(sel_a, s0)
    k1l = _pick(sel_a, s1)
    k2h = _pick(sel_b, s0)
    k2l = _pick(sel_b, s1)

    rg128 = lax.broadcasted_iota(jnp.uint32, (_TC_ROWS, 128), 0) + jnp.uint32(_SC_ROWS)
    b128 = (rg128 * jnp.uint32(21846)) >> jnp.uint32(16)
    f128 = (rg128 - jnp.uint32(3) * b128).astype(jnp.int32)
    u0, u1 = _threefry2x32(k1h, k1l, jnp.zeros((_TC_ROWS, 128), jnp.uint32), b128, _shr_u)
    n1 = ((u0 ^ u1) >> jnp.uint32(22)).astype(jnp.int32)
    n_all = jnp.where(f128 == 0, n1, jnp.where(f128 == 1, _N2, _P - n1))
    n = n_all[:, :1]

    r_i = lax.broadcasted_iota(jnp.uint32, (_TC_ROWS, _P), 0) + jnp.uint32(_SC_ROWS)
    j_i = lax.broadcasted_iota(jnp.uint32, (_TC_ROWS, _P), 1)
    cnt = r_i * jnp.uint32(_P) + j_i
    y0, y1 = _threefry2x32(k2h, k2l, jnp.zeros((_TC_ROWS, _P), jnp.uint32), cnt, _shr_u)
    m = ((y0 ^ y1) >> jnp.uint32(9)).astype(jnp.int32)

    t, rem = _tc_select_rank(m, 23, n, None)
    eq = m == t
    jj = lax.broadcasted_iota(jnp.int32, (_TC_ROWS, _P), 1)
    j_thresh, _ = _tc_select_rank(jj, 10, rem, eq)
    mask = (m < t) | (eq & (jj <= j_thresh) & (n > 0))
    out_ref[...] = mask


def kernel(x):
    sc_fn = functools.partial(
        pl.kernel,
        out_type=jax.ShapeDtypeStruct((_SC_ROWS * _P,), jnp.int32),
        mesh=plsc.VectorSubcoreMesh(core_axis_name="c", subcore_axis_name="s"),
        compiler_params=pltpu.CompilerParams(needs_layout_passes=False),
        scratch_types=[
            pltpu.VMEM((_P,), jnp.int32),
            pltpu.VMEM((_P,), jnp.int32),
        ],
    )(_sc_body)
    sc_flat = sc_fn()
    tc_mask = pl.pallas_call(
        _tc_body,
        out_shape=jax.ShapeDtypeStruct((_TC_ROWS, _P), jnp.bool_),
    )()
    sc_mask = sc_flat.reshape(_SC_ROWS, _P).astype(jnp.bool_)
    return jnp.concatenate([sc_mask, tc_mask], axis=0).reshape(_B, _F * _P)


# speedup vs baseline: 1.3386x; 1.0689x over previous
import functools

import jax
import jax.numpy as jnp
from jax import lax
from jax.experimental import pallas as pl
from jax.experimental.pallas import tpu as pltpu
from jax.experimental.pallas import tpu_sc as plsc

_B = 32
_F = 3
_P = 1024
_R = _B * _F
_N2 = int(0.9 * _P)
_NV = _P // 16
_SC_ROWS = 32
_TC_ROWS = _R - _SC_ROWS


def _threefry2x32(ks0, ks1, x0, x1, shr):
    ks2 = ks0 ^ ks1 ^ _c(ks0, 0x1BD11BDA)
    ks = (ks0, ks1, ks2)
    rots = ((13, 15, 26, 6), (17, 29, 16, 24))
    x0 = x0 + ks0
    x1 = x1 + ks1
    for g in range(5):
        for r in rots[g % 2]:
            x0 = x0 + x1
            x1 = (x1 << _c(x1, r)) | shr(x1, 32 - r)
            x1 = x1 ^ x0
        x0 = x0 + ks[(g + 1) % 3]
        x1 = x1 + ks[(g + 2) % 3] + _c(x1, g + 1)
    return x0, x1


def _c(like, v):
    return jnp.asarray(v, dtype=like.dtype if hasattr(like, "dtype") else jnp.int32)


def _shr_u(x, r):
    return x >> jnp.uint32(r)


def _shr_s(x, r):
    return lax.shift_right_logical(x, jnp.int32(r))


def _sc_body(out_hbm, m_v, o_v):
    cid = lax.axis_index("c")
    sid = lax.axis_index("s")
    w = sid * 2 + cid

    zero = jnp.int32(0)
    one = jnp.int32(1)
    vzero = jnp.zeros((16,), jnp.int32)
    k1h, k1l = _threefry2x32(zero, jnp.int32(42), zero, zero, _shr_s)
    k2h, k2l = _threefry2x32(zero, jnp.int32(42), zero, one, _shr_s)

    r = w
    b = lax.shift_right_logical(r * jnp.int32(21846), jnp.int32(16))
    f = r - 3 * b
    u0, u1 = _threefry2x32(k1h, k1l, zero, b, _shr_s)
    n1 = lax.shift_right_logical(u0 ^ u1, jnp.int32(22))
    n = jnp.where(f == 0, n1, jnp.where(f == 1, jnp.int32(_N2), jnp.int32(_P) - n1))

    def gen(v, _cc):
        lane = lax.iota(jnp.int32, 16)
        cnt = r * jnp.int32(_P) + v * jnp.int32(16) + lane
        y0, y1 = _threefry2x32(k2h, k2l, zero, cnt, _shr_s)
        m_v[pl.ds(v * 16, 16)] = lax.shift_right_logical(y0 ^ y1, jnp.int32(9))
        return _cc
    lax.fori_loop(0, _NV, gen, zero, unroll=8)

    def not_done(st):
        bit, pref, rem, cand = st
        return (bit >= 0) & (cand > 1)

    def level(st):
        bit, pref, rem, cand = st
        tgt = pref << one

        def cpass(v, acc):
            mv = m_v[pl.ds(v * 16, 16)]
            hit = lax.shift_right_logical(mv, bit) == tgt
            return acc + hit.astype(jnp.int32)
        accv = lax.fori_loop(0, _NV, cpass, vzero, unroll=8)
        c0 = jnp.sum(accv)
        go1 = rem > c0
        return (bit - one,
                tgt | go1.astype(jnp.int32),
                rem - jnp.where(go1, c0, zero),
                jnp.where(go1, cand - c0, c0))

    bit, pref, rem, cand = lax.while_loop(
        not_done, level, (jnp.int32(22), zero, n, jnp.int32(_P)))

    def fpass(v, acc):
        mv = m_v[pl.ds(v * 16, 16)]
        hit = lax.shift_right_logical(mv, bit + one) == pref
        return acc + jnp.where(hit, mv, vzero)
    accf = lax.fori_loop(0, _NV, fpass, vzero, unroll=8)
    t = jnp.where(cand == one, jnp.sum(accf), pref)

    def tpass(v, accs):
        acc_e, acc_j = accs
        lane = lax.iota(jnp.int32, 16)
        mv = m_v[pl.ds(v * 16, 16)]
        eq = (mv == t).astype(jnp.int32)
        jj1 = v * jnp.int32(16) + lane + one
        return (acc_e + eq, acc_j + eq * jj1)
    acc_e, acc_j = lax.fori_loop(0, _NV, tpass, (vzero, vzero), unroll=8)
    c_eq = jnp.sum(acc_e)

    def unique_j(_):
        return jnp.sum(acc_j) - one

    def radix_j(_):
        prefj = zero
        remj = rem
        for bit in range(9, -1, -1):
            tgtj = prefj << one

            def jpass(v, acc, _bit=bit, _tgtj=tgtj):
                lane = lax.iota(jnp.int32, 16)
                mv = m_v[pl.ds(v * 16, 16)]
                jj = v * jnp.int32(16) + lane
                hit = (mv == t) & (lax.shift_right_logical(jj, jnp.int32(_bit)) == _tgtj)
                return acc + hit.astype(jnp.int32)
            accj = lax.fori_loop(0, _NV, jpass, vzero, unroll=8)
            cj = jnp.sum(accj)
            go = remj > cj
            prefj = tgtj | go.astype(jnp.int32)
            remj = remj - jnp.where(go, cj, zero)
        return prefj

    j_t = lax.cond(c_eq == one, unique_j, radix_j, zero)
    t_eff = jnp.where(n > 0, t, jnp.int32(-1))
    j_eff = jnp.where(n > 0, j_t, jnp.int32(-1))

    def emit(v, _cc):
        lane = lax.iota(jnp.int32, 16)
        mv = m_v[pl.ds(v * 16, 16)]
        jj = v * jnp.int32(16) + lane
        mask = (mv < t_eff) | ((mv == t_eff) & (jj <= j_eff))
        o_v[pl.ds(v * 16, 16)] = mask.astype(jnp.int32)
        return _cc
    lax.fori_loop(0, _NV, emit, zero, unroll=8)
    pltpu.sync_copy(o_v, out_hbm.at[pl.ds(r * _P, _P)])


def _tc_select_rank(keys, nbits, rem, cand):
    pref = jnp.zeros_like(rem)
    for bit in range(nbits - 1, -1, -1):
        ms = keys >> bit
        match = (ms >> 1) == pref
        if cand is not None:
            match = match & cand
        in0 = match & ((ms & 1) == 0)
        c0 = jnp.sum(in0.astype(jnp.int32), axis=1, keepdims=True)
        go1 = rem > c0
        pref = (pref << 1) | go1.astype(jnp.int32)
        rem = rem - jnp.where(go1, c0, 0)
    return pref, rem


def _tc_body(out_ref):
    col8 = lax.broadcasted_iota(jnp.uint32, (8, 128), 1)
    row8 = lax.broadcasted_iota(jnp.uint32, (8, 128), 0)
    s0, s1 = _threefry2x32(jnp.uint32(0), jnp.uint32(42),
                           jnp.zeros((8, 128), jnp.uint32), col8, _shr_u)
    top = row8 == jnp.uint32(0)
    sel_a = top & (col8 == jnp.uint32(0))
    sel_b = top & (col8 == jnp.uint32(1))

    def _pick(sel, v):
        vi = lax.bitcast_convert_type(v, jnp.int32)
        s = jnp.sum(jnp.where(sel, vi, 0))
        return lax.bitcast_convert_type(s, jnp.uint32)

    k1h = _pick(sel_a, s0)
    k1l = _pick(sel_a, s1)
    k2h = _pick(sel_b, s0)
    k2l = _pick(sel_b, s1)

    rg128 = lax.broadcasted_iota(jnp.uint32, (_TC_ROWS, 128), 0) + jnp.uint32(_SC_ROWS)
    b128 = (rg128 * jnp.uint32(21846)) >> jnp.uint32(16)
    f128 = (rg128 - jnp.uint32(3) * b128).astype(jnp.int32)
    u0, u1 = _threefry2x32(k1h, k1l, jnp.zeros((_TC_ROWS, 128), jnp.uint32), b128, _shr_u)
    n1 = ((u0 ^ u1) >> jnp.uint32(22)).astype(jnp.int32)
    n_all = jnp.where(f128 == 0, n1, jnp.where(f128 == 1, _N2, _P - n1))
    n = n_all[:, :1]

    r_i = lax.broadcasted_iota(jnp.uint32, (_TC_ROWS, _P), 0) + jnp.uint32(_SC_ROWS)
    j_i = lax.broadcasted_iota(jnp.uint32, (_TC_ROWS, _P), 1)
    cnt = r_i * jnp.uint32(_P) + j_i
    y0, y1 = _threefry2x32(k2h, k2l, jnp.zeros((_TC_ROWS, _P), jnp.uint32), cnt, _shr_u)
    m = ((y0 ^ y1) >> jnp.uint32(9)).astype(jnp.int32)

    t, rem = _tc_select_rank(m, 23, n, None)
    eq = m == t
    jj = lax.broadcasted_iota(jnp.int32, (_TC_ROWS, _P), 1)
    j_thresh, _ = _tc_select_rank(jj, 10, rem, eq)
    mask = (m < t) | (eq & (jj <= j_thresh) & (n > 0))
    out_ref[...] = mask


def kernel(x):
    sc_fn = functools.partial(
        pl.kernel,
        out_type=jax.ShapeDtypeStruct((_SC_ROWS * _P,), jnp.int32),
        mesh=plsc.VectorSubcoreMesh(core_axis_name="c", subcore_axis_name="s"),
        compiler_params=pltpu.CompilerParams(needs_layout_passes=False),
        scratch_types=[
            pltpu.VMEM((_P,), jnp.int32),
            pltpu.VMEM((_P,), jnp.int32),
        ],
    )(_sc_body)
    sc_flat = sc_fn()
    tc_mask = pl.pallas_call(
        _tc_body,
        out_shape=jax.ShapeDtypeStruct((_TC_ROWS, _P), jnp.bool_),
    )()
    sc_mask = sc_flat.reshape(_SC_ROWS, _P).astype(jnp.bool_)
    return jnp.concatenate([sc_mask, tc_mask], axis=0).reshape(_B, _F * _P)


# speedup vs baseline: 1.3440x; 1.0040x over previous
import functools

import jax
import jax.numpy as jnp
from jax import lax
from jax.experimental import pallas as pl
from jax.experimental.pallas import tpu as pltpu
from jax.experimental.pallas import tpu_sc as plsc

_B = 32
_F = 3
_P = 1024
_R = _B * _F
_N2 = int(0.9 * _P)
_NV = _P // 16
_SC_ROWS = 32
_TC_ROWS = _R - _SC_ROWS


def _threefry2x32(ks0, ks1, x0, x1, shr):
    ks2 = ks0 ^ ks1 ^ _c(ks0, 0x1BD11BDA)
    ks = (ks0, ks1, ks2)
    rots = ((13, 15, 26, 6), (17, 29, 16, 24))
    x0 = x0 + ks0
    x1 = x1 + ks1
    for g in range(5):
        for r in rots[g % 2]:
            x0 = x0 + x1
            x1 = (x1 << _c(x1, r)) | shr(x1, 32 - r)
            x1 = x1 ^ x0
        x0 = x0 + ks[(g + 1) % 3]
        x1 = x1 + ks[(g + 2) % 3] + _c(x1, g + 1)
    return x0, x1


def _c(like, v):
    return jnp.asarray(v, dtype=like.dtype if hasattr(like, "dtype") else jnp.int32)


def _shr_u(x, r):
    return x >> jnp.uint32(r)


def _shr_s(x, r):
    return lax.shift_right_logical(x, jnp.int32(r))


def _sc_body(out_hbm, m_v, o_v):
    cid = lax.axis_index("c")
    sid = lax.axis_index("s")
    w = sid * 2 + cid

    zero = jnp.int32(0)
    one = jnp.int32(1)
    vzero = jnp.zeros((16,), jnp.int32)
    k1h, k1l = _threefry2x32(zero, jnp.int32(42), zero, zero, _shr_s)
    k2h, k2l = _threefry2x32(zero, jnp.int32(42), zero, one, _shr_s)

    r = w
    b = lax.shift_right_logical(r * jnp.int32(21846), jnp.int32(16))
    f = r - 3 * b
    u0, u1 = _threefry2x32(k1h, k1l, zero, b, _shr_s)
    n1 = lax.shift_right_logical(u0 ^ u1, jnp.int32(22))
    n = jnp.where(f == 0, n1, jnp.where(f == 1, jnp.int32(_N2), jnp.int32(_P) - n1))

    def gen(v, _cc):
        lane = lax.iota(jnp.int32, 16)
        cnt = r * jnp.int32(_P) + v * jnp.int32(16) + lane
        y0, y1 = _threefry2x32(k2h, k2l, zero, cnt, _shr_s)
        m_v[pl.ds(v * 16, 16)] = lax.shift_right_logical(y0 ^ y1, jnp.int32(9))
        return _cc
    lax.fori_loop(0, _NV, gen, zero, unroll=16)

    def not_done(st):
        bit, pref, rem, cand = st
        return (bit >= 0) & (cand > 1)

    def level(st):
        bit, pref, rem, cand = st
        tgt = pref << one

        def cpass(v, acc):
            mv = m_v[pl.ds(v * 16, 16)]
            hit = lax.shift_right_logical(mv, bit) == tgt
            return acc + hit.astype(jnp.int32)
        accv = lax.fori_loop(0, _NV, cpass, vzero, unroll=8)
        c0 = jnp.sum(accv)
        go1 = rem > c0
        return (bit - one,
                tgt | go1.astype(jnp.int32),
                rem - jnp.where(go1, c0, zero),
                jnp.where(go1, cand - c0, c0))

    bit, pref, rem, cand = lax.while_loop(
        not_done, level, (jnp.int32(22), zero, n, jnp.int32(_P)))

    def fpass(v, accs):
        acc_m, acc_j = accs
        lane = lax.iota(jnp.int32, 16)
        mv = m_v[pl.ds(v * 16, 16)]
        hit = lax.shift_right_logical(mv, bit + one) == pref
        jj1 = v * jnp.int32(16) + lane + one
        return (acc_m + jnp.where(hit, mv, vzero),
                acc_j + jnp.where(hit, jj1, vzero))
    acc_m, acc_j = lax.fori_loop(0, _NV, fpass, (vzero, vzero), unroll=8)
    t = jnp.where(cand == one, jnp.sum(acc_m), pref)

    def unique_j(_):
        return jnp.sum(acc_j) - one

    def radix_j(_):
        prefj = zero
        remj = rem
        for bit in range(9, -1, -1):
            tgtj = prefj << one

            def jpass(v, acc, _bit=bit, _tgtj=tgtj):
                lane = lax.iota(jnp.int32, 16)
                mv = m_v[pl.ds(v * 16, 16)]
                jj = v * jnp.int32(16) + lane
                hit = (mv == t) & (lax.shift_right_logical(jj, jnp.int32(_bit)) == _tgtj)
                return acc + hit.astype(jnp.int32)
            accj = lax.fori_loop(0, _NV, jpass, vzero, unroll=8)
            cj = jnp.sum(accj)
            go = remj > cj
            prefj = tgtj | go.astype(jnp.int32)
            remj = remj - jnp.where(go, cj, zero)
        return prefj

    j_t = lax.cond(cand == one, unique_j, radix_j, zero)
    t_eff = jnp.where(n > 0, t, jnp.int32(-1))
    j_eff = jnp.where(n > 0, j_t, jnp.int32(-1))

    def emit(v, _cc):
        lane = lax.iota(jnp.int32, 16)
        mv = m_v[pl.ds(v * 16, 16)]
        jj = v * jnp.int32(16) + lane
        mask = (mv < t_eff) | ((mv == t_eff) & (jj <= j_eff))
        o_v[pl.ds(v * 16, 16)] = mask.astype(jnp.int32)
        return _cc
    lax.fori_loop(0, _NV, emit, zero, unroll=8)
    pltpu.sync_copy(o_v, out_hbm.at[pl.ds(r * _P, _P)])


def _tc_select_rank(keys, nbits, rem, cand):
    pref = jnp.zeros_like(rem)
    for bit in range(nbits - 1, -1, -1):
        ms = keys >> bit
        match = (ms >> 1) == pref
        if cand is not None:
            match = match & cand
        in0 = match & ((ms & 1) == 0)
        c0 = jnp.sum(in0.astype(jnp.int32), axis=1, keepdims=True)
        go1 = rem > c0
        pref = (pref << 1) | go1.astype(jnp.int32)
        rem = rem - jnp.where(go1, c0, 0)
    return pref, rem


def _tc_body(out_ref):
    col8 = lax.broadcasted_iota(jnp.uint32, (8, 128), 1)
    row8 = lax.broadcasted_iota(jnp.uint32, (8, 128), 0)
    s0, s1 = _threefry2x32(jnp.uint32(0), jnp.uint32(42),
                           jnp.zeros((8, 128), jnp.uint32), col8, _shr_u)
    top = row8 == jnp.uint32(0)
    sel_a = top & (col8 == jnp.uint32(0))
    sel_b = top & (col8 == jnp.uint32(1))

    def _pick(sel, v):
        vi = lax.bitcast_convert_type(v, jnp.int32)
        s = jnp.sum(jnp.where(sel, vi, 0))
        return lax.bitcast_convert_type(s, jnp.uint32)

    k1h = _pick(sel_a, s0)
    k1l = _pick(sel_a, s1)
    k2h = _pick(sel_b, s0)
    k2l = _pick(sel_b, s1)

    rg128 = lax.broadcasted_iota(jnp.uint32, (_TC_ROWS, 128), 0) + jnp.uint32(_SC_ROWS)
    b128 = (rg128 * jnp.uint32(21846)) >> jnp.uint32(16)
    f128 = (rg128 - jnp.uint32(3) * b128).astype(jnp.int32)
    u0, u1 = _threefry2x32(k1h, k1l, jnp.zeros((_TC_ROWS, 128), jnp.uint32), b128, _shr_u)
    n1 = ((u0 ^ u1) >> jnp.uint32(22)).astype(jnp.int32)
    n_all = jnp.where(f128 == 0, n1, jnp.where(f128 == 1, _N2, _P - n1))
    n = n_all[:, :1]

    r_i = lax.broadcasted_iota(jnp.uint32, (_TC_ROWS, _P), 0) + jnp.uint32(_SC_ROWS)
    j_i = lax.broadcasted_iota(jnp.uint32, (_TC_ROWS, _P), 1)
    cnt = r_i * jnp.uint32(_P) + j_i
    y0, y1 = _threefry2x32(k2h, k2l, jnp.zeros((_TC_ROWS, _P), jnp.uint32), cnt, _shr_u)
    m = ((y0 ^ y1) >> jnp.uint32(9)).astype(jnp.int32)

    t, rem = _tc_select_rank(m, 23, n, None)
    eq = m == t
    jj = lax.broadcasted_iota(jnp.int32, (_TC_ROWS, _P), 1)
    j_thresh, _ = _tc_select_rank(jj, 10, rem, eq)
    mask = (m < t) | (eq & (jj <= j_thresh) & (n > 0))
    out_ref[...] = mask


def kernel(x):
    sc_fn = functools.partial(
        pl.kernel,
        out_type=jax.ShapeDtypeStruct((_SC_ROWS * _P,), jnp.int32),
        mesh=plsc.VectorSubcoreMesh(core_axis_name="c", subcore_axis_name="s"),
        compiler_params=pltpu.CompilerParams(needs_layout_passes=False),
        scratch_types=[
            pltpu.VMEM((_P,), jnp.int32),
            pltpu.VMEM((_P,), jnp.int32),
        ],
    )(_sc_body)
    sc_flat = sc_fn()
    tc_mask = pl.pallas_call(
        _tc_body,
        out_shape=jax.ShapeDtypeStruct((_TC_ROWS, _P), jnp.bool_),
    )()
    sc_mask = sc_flat.reshape(_SC_ROWS, _P).astype(jnp.bool_)
    return jnp.concatenate([sc_mask, tc_mask], axis=0).reshape(_B, _F * _P)
